# speedup vs baseline: 1.6819x; 1.6819x over previous
import functools

import jax
import jax.numpy as jnp
from jax import lax
from jax.experimental import pallas as pl
from jax.experimental.pallas import tpu as pltpu
from jax.experimental.pallas import tpu_sc as plsc

VOCAB = 100000
HID = 128
MAXPOS = 2048
B = 4
S = 2048
NTOK = B * S

NC = 2
NS = 16
NW = NC * NS
TOK_PER_W = NTOK // NW
IDX_CHUNK = 128
N_CHUNKS = TOK_PER_W // IDX_CHUNK


def _sc_gather(table, idx2d):
  mesh = plsc.VectorSubcoreMesh(
      core_axis_name="c", subcore_axis_name="s", num_cores=NC, num_subcores=NS
  )

  @functools.partial(
      pl.kernel,
      mesh=mesh,
      out_type=jax.ShapeDtypeStruct((NTOK, HID), jnp.float32),
      scratch_types=[
          pltpu.VMEM((N_CHUNKS, IDX_CHUNK), jnp.int32),
          pltpu.VMEM((TOK_PER_W, HID), jnp.float32),
          pltpu.SemaphoreType.DMA,
      ],
  )
  def gather_kernel(table_hbm, idx_hbm, out_hbm, idx_v, rows_v, sem):
    wid = lax.axis_index("s") * NC + lax.axis_index("c")
    row0 = wid * N_CHUNKS
    pltpu.sync_copy(idx_hbm.at[pl.ds(row0, N_CHUNKS)], idx_v)
    copies = []
    for j in range(N_CHUNKS):
      copies.append(
          pltpu.async_copy(
              table_hbm.at[idx_v.at[j]],
              rows_v.at[pl.ds(j * IDX_CHUNK, IDX_CHUNK)],
              sem,
          )
      )
    for c in copies:
      c.wait()
    pltpu.sync_copy(rows_v, out_hbm.at[pl.ds(wid * TOK_PER_W, TOK_PER_W)])

  return gather_kernel(table, idx2d)


TC_BLK = 512
POS_BLKS = S // TC_BLK


def _tc_body(g_ref, pos_ref, t_ref, wt_ref, lnw_ref, lnb_ref, o_ref):
  x = g_ref[...] + pos_ref[...]
  t = t_ref[...]
  typ = jnp.where(t > 0.5, wt_ref[1:2, :], wt_ref[0:1, :])
  x = x + typ
  mean = jnp.mean(x, axis=-1, keepdims=True)
  xc = x - mean
  var = jnp.mean(xc * xc, axis=-1, keepdims=True)
  o_ref[...] = xc * (lnw_ref[...] * lax.rsqrt(var + 1e-5)) + lnb_ref[...]


def _tc_ln(gathered, t_f32, W_pos, W_type, ln_w, ln_b):
  grid = NTOK // TC_BLK
  return pl.pallas_call(
      _tc_body,
      grid=(grid,),
      in_specs=[
          pl.BlockSpec((TC_BLK, HID), lambda j: (j, 0)),
          pl.BlockSpec((TC_BLK, HID), lambda j: (j % POS_BLKS, 0)),
          pl.BlockSpec((TC_BLK, 1), lambda j: (j, 0)),
          pl.BlockSpec((2, HID), lambda j: (0, 0)),
          pl.BlockSpec((1, HID), lambda j: (0, 0)),
          pl.BlockSpec((1, HID), lambda j: (0, 0)),
      ],
      out_specs=pl.BlockSpec((TC_BLK, HID), lambda j: (j, 0)),
      out_shape=jax.ShapeDtypeStruct((NTOK, HID), jnp.float32),
  )(gathered, W_pos, t_f32, W_type, ln_w, ln_b)


def kernel(input_ids, token_type_ids, W_tok, W_pos, W_type, ln_w, ln_b):
  idx2d = input_ids.astype(jnp.int32).reshape(NTOK // IDX_CHUNK, IDX_CHUNK)
  gathered = _sc_gather(W_tok, idx2d)
  t_f32 = token_type_ids.astype(jnp.float32).reshape(NTOK, 1)
  out = _tc_ln(
      gathered,
      t_f32,
      W_pos,
      W_type,
      ln_w.reshape(1, HID),
      ln_b.reshape(1, HID),
  )
  return out.reshape(B, S, HID)
